# Initial kernel scaffold; baseline (speedup 1.0000x reference)
import jax
import jax.numpy as jnp
from jax.experimental import pallas as pl


def kernel(node_features, neighbor_indices, relation_kernels, self_kernel, bias):
    raise NotImplementedError("write your pallas kernel here")


# speedup vs baseline: 2.7670x; 2.7670x over previous
import functools

import jax
import jax.numpy as jnp
from jax import lax
from jax.experimental import pallas as pl
from jax.experimental.pallas import tpu as pltpu
from jax.experimental.pallas import tpu_sc as plsc

_N = 10000
_N_PAD = 10240
_R = 3
_K = 16
_D = 128
_NW = 32
_ROWS = _R * _N_PAD
_ROWS_PER_W = _ROWS // _NW
_C = 8
_CHUNKS = _ROWS_PER_W // _C


def _sc_body(table_hbm, idx_hbm, out_hbm, idx_v, rows_v, acc_v, sem):
    wid = lax.axis_index("s") * 2 + lax.axis_index("c")
    base = wid * _ROWS_PER_W

    @pl.loop(0, _CHUNKS)
    def _chunk(c):
        row0 = base + c * _C
        pltpu.sync_copy(idx_hbm.at[pl.ds(row0 * _K, _C * _K)], idx_v)
        pltpu.async_copy(table_hbm.at[idx_v], rows_v, sem).wait()
        for i in range(_C):
            for j in range(_D // 16):
                v = rows_v[i * _K, pl.ds(j * 16, 16)]
                for kk in range(1, _K):
                    v = v + rows_v[i * _K + kk, pl.ds(j * 16, 16)]
                acc_v[i, pl.ds(j * 16, 16)] = v
        pltpu.sync_copy(acc_v, out_hbm.at[pl.ds(row0, _C), :])


@jax.jit
def _sc_aggregate(table, idx_flat):
    mesh = plsc.VectorSubcoreMesh(core_axis_name="c", subcore_axis_name="s")
    k = functools.partial(
        pl.kernel,
        out_type=jax.ShapeDtypeStruct((_ROWS, _D), jnp.float32),
        mesh=mesh,
        scratch_types=[
            pltpu.VMEM((_C * _K,), jnp.int32),
            pltpu.VMEM((_C * _K, _D), jnp.float32),
            pltpu.VMEM((_C, _D), jnp.float32),
            pltpu.SemaphoreType.DMA,
        ],
    )(_sc_body)
    return k(table, idx_flat)


def _tc_body(agg_ref, x_ref, wr_ref, ws_ref, b_ref, o_ref):
    acc = jnp.dot(x_ref[...], ws_ref[...], preferred_element_type=jnp.float32)
    for r in range(_R):
        acc = acc + jnp.dot(agg_ref[r], wr_ref[r], preferred_element_type=jnp.float32)
    o_ref[...] = jnp.maximum(acc + b_ref[...], 0.0)


def _tc_combine(agg, x_pad, wr, ws, bias2d):
    bn = 512
    return pl.pallas_call(
        _tc_body,
        grid=(_N_PAD // bn,),
        in_specs=[
            pl.BlockSpec((_R, bn, _D), lambda i: (0, i, 0)),
            pl.BlockSpec((bn, _D), lambda i: (i, 0)),
            pl.BlockSpec((_R, _D, _D), lambda i: (0, 0, 0)),
            pl.BlockSpec((_D, _D), lambda i: (0, 0)),
            pl.BlockSpec((1, _D), lambda i: (0, 0)),
        ],
        out_specs=pl.BlockSpec((bn, _D), lambda i: (i, 0)),
        out_shape=jax.ShapeDtypeStruct((_N_PAD, _D), jnp.float32),
    )(agg, x_pad, wr, ws, bias2d)


def kernel(node_features, neighbor_indices, relation_kernels, self_kernel, bias):
    b, n, d = node_features.shape
    x = node_features[0]
    table = jnp.concatenate([jnp.zeros((1, d), x.dtype), x], axis=0)
    idx = neighbor_indices[0].astype(jnp.int32)
    idx = jnp.pad(idx, ((0, 0), (0, _N_PAD - n), (0, 0)))
    agg = _sc_aggregate(table, idx.reshape(-1))
    agg = agg.reshape(_R, _N_PAD, _D)
    x_pad = jnp.pad(x, ((0, _N_PAD - n), (0, 0)))
    wr = relation_kernels * (1.0 / _K)
    out = _tc_combine(agg, x_pad, wr, self_kernel, bias.reshape(1, _D))
    return out[None, :n, :]


# speedup vs baseline: 3.2569x; 1.1771x over previous
import functools

import jax
import jax.numpy as jnp
from jax import lax
from jax.experimental import pallas as pl
from jax.experimental.pallas import tpu as pltpu
from jax.experimental.pallas import tpu_sc as plsc

_N = 10000
_N_PAD = 10240
_R = 3
_K = 16
_D = 128
_NW = 32
_ROWS = _R * _N_PAD
_ROWS_PER_W = _ROWS // _NW
_C = 8
_CHUNKS = _ROWS_PER_W // _C


_NBUF = 2


def _sc_body(table_hbm, idx_hbm, out_hbm, idx_v, rows_v, acc_v, *sems):
    gsems = sems[:_NBUF]
    osems = sems[_NBUF:]
    wid = lax.axis_index("s") * 2 + lax.axis_index("c")
    base = wid * _ROWS_PER_W
    ibase = wid * _CHUNKS

    pltpu.sync_copy(idx_hbm.at[pl.ds(ibase, _CHUNKS), :], idx_v)
    pltpu.async_copy(table_hbm.at[idx_v.at[0]], rows_v.at[0], gsems[0])

    @pl.loop(0, _CHUNKS, step=_NBUF)
    def _c0(c0):
        for b in range(_NBUF):
            c = c0 + b
            nb = (b + 1) % _NBUF

            @pl.when(c + 1 < _CHUNKS)
            def _():
                pltpu.async_copy(
                    table_hbm.at[idx_v.at[c + 1]], rows_v.at[nb], gsems[nb])

            pltpu.make_async_copy(
                table_hbm.at[idx_v.at[c]], rows_v.at[b], gsems[b]).wait()

            @pl.when(c >= _NBUF)
            def _():
                pltpu.make_async_copy(
                    acc_v.at[b],
                    out_hbm.at[pl.ds(base + (c - _NBUF) * _C, _C), :],
                    osems[b]).wait()

            for i in range(_C):
                for j in range(_D // 16):
                    v = rows_v[b, i * _K, pl.ds(j * 16, 16)]
                    for kk in range(1, _K):
                        v = v + rows_v[b, i * _K + kk, pl.ds(j * 16, 16)]
                    acc_v[b, i, pl.ds(j * 16, 16)] = v

            pltpu.async_copy(
                acc_v.at[b], out_hbm.at[pl.ds(base + c * _C, _C), :], osems[b])

    for b in range(_NBUF):
        pltpu.make_async_copy(
            acc_v.at[b], out_hbm.at[pl.ds(base + b * _C, _C), :],
            osems[b]).wait()


@jax.jit
def _sc_aggregate(table, idx2d):
    mesh = plsc.VectorSubcoreMesh(core_axis_name="c", subcore_axis_name="s")
    k = functools.partial(
        pl.kernel,
        out_type=jax.ShapeDtypeStruct((_ROWS, _D), jnp.float32),
        mesh=mesh,
        scratch_types=[
            pltpu.VMEM((_CHUNKS, _C * _K), jnp.int32),
            pltpu.VMEM((_NBUF, _C * _K, _D), jnp.float32),
            pltpu.VMEM((_NBUF, _C, _D), jnp.float32),
        ] + [pltpu.SemaphoreType.DMA] * (2 * _NBUF),
    )(_sc_body)
    return k(table, idx2d)


def _tc_body(agg_ref, x_ref, wr_ref, ws_ref, b_ref, o_ref):
    acc = jnp.dot(x_ref[...], ws_ref[...], preferred_element_type=jnp.float32)
    for r in range(_R):
        acc = acc + jnp.dot(agg_ref[r], wr_ref[r], preferred_element_type=jnp.float32)
    o_ref[...] = jnp.maximum(acc + b_ref[...], 0.0)


def _tc_combine(agg, x_pad, wr, ws, bias2d):
    bn = 512
    return pl.pallas_call(
        _tc_body,
        grid=(_N_PAD // bn,),
        in_specs=[
            pl.BlockSpec((_R, bn, _D), lambda i: (0, i, 0)),
            pl.BlockSpec((bn, _D), lambda i: (i, 0)),
            pl.BlockSpec((_R, _D, _D), lambda i: (0, 0, 0)),
            pl.BlockSpec((_D, _D), lambda i: (0, 0)),
            pl.BlockSpec((1, _D), lambda i: (0, 0)),
        ],
        out_specs=pl.BlockSpec((bn, _D), lambda i: (i, 0)),
        out_shape=jax.ShapeDtypeStruct((_N_PAD, _D), jnp.float32),
    )(agg, x_pad, wr, ws, bias2d)


def kernel(node_features, neighbor_indices, relation_kernels, self_kernel, bias):
    b, n, d = node_features.shape
    x = node_features[0]
    table = jnp.concatenate([jnp.zeros((1, d), x.dtype), x], axis=0)
    idx = neighbor_indices[0].astype(jnp.int32)
    idx = jnp.pad(idx, ((0, 0), (0, _N_PAD - n), (0, 0)))
    agg = _sc_aggregate(table, idx.reshape(-1, _C * _K))
    agg = agg.reshape(_R, _N_PAD, _D)
    x_pad = jnp.pad(x, ((0, _N_PAD - n), (0, 0)))
    wr = relation_kernels * (1.0 / _K)
    out = _tc_combine(agg, x_pad, wr, self_kernel, bias.reshape(1, _D))
    return out[None, :n, :]
